# baseline (speedup 1.0000x reference)
import functools

import jax
import jax.numpy as jnp
from jax import lax
from jax.experimental import pallas as pl
from jax.experimental.pallas import tpu as pltpu

B = 2
W_, H_, T_ = 384, 256, 20
L, F, TS = 8, 4, 2 ** 19
P = 4
N_EV = 160000


def _ln(x, g, b, eps=1e-6):
    m = x.mean(axis=-1, keepdims=True)
    v = ((x - m) ** 2).mean(axis=-1, keepdims=True)
    return (x - m) / jnp.sqrt(v + eps) * g + b


def _conv(x, w, b, stride, pad, groups=1):
    y = lax.conv_general_dilated(
        x, w, window_strides=(stride, stride),
        padding=[(pad, pad), (pad, pad)],
        dimension_numbers=('NHWC', 'HWIO', 'NHWC'),
        feature_group_count=groups)
    return y + b


def _resolutions():
    coarse = jnp.log(jnp.array([16.0, 16.0, 4.0]))
    fine = jnp.log(jnp.array([384.0, 256.0, 16.0]))
    lev = jnp.arange(L, dtype=jnp.float32)[:, None]
    return jnp.exp(coarse[None] + lev * (fine - coarse)[None] / (L - 1)).astype(jnp.int32)


def _mlp_body(x_ref, y_ref, g_ref, b_ref, w1_ref, b1_ref, w2_ref, b2_ref,
              gamma_ref, o_ref):
    y = y_ref[...]
    m = y.mean(axis=-1, keepdims=True)
    v = ((y - m) ** 2).mean(axis=-1, keepdims=True)
    y = (y - m) / jnp.sqrt(v + 1e-6) * g_ref[...] + b_ref[...]
    h = jnp.dot(y, w1_ref[...], preferred_element_type=jnp.float32) + b1_ref[...]
    h = jax.nn.gelu(h)
    y = jnp.dot(h, w2_ref[...], preferred_element_type=jnp.float32) + b2_ref[...]
    o_ref[...] = x_ref[...] + y * gamma_ref[...]


def _mlp_block(x, y, g, b, w1, b1, w2, b2, gamma):
    n, h_, w_, c = x.shape
    rows = n * h_ * w_
    xf = x.reshape(rows, c)
    yf = y.reshape(rows, c)
    blk = 2048
    grid = rows // blk
    out = pl.pallas_call(
        _mlp_body,
        out_shape=jax.ShapeDtypeStruct((rows, c), jnp.float32),
        grid=(grid,),
        in_specs=[
            pl.BlockSpec((blk, c), lambda i: (i, 0)),
            pl.BlockSpec((blk, c), lambda i: (i, 0)),
            pl.BlockSpec((c,), lambda i: (0,)),
            pl.BlockSpec((c,), lambda i: (0,)),
            pl.BlockSpec(w1.shape, lambda i: (0, 0)),
            pl.BlockSpec(b1.shape, lambda i: (0,)),
            pl.BlockSpec(w2.shape, lambda i: (0, 0)),
            pl.BlockSpec(b2.shape, lambda i: (0,)),
            pl.BlockSpec((c,), lambda i: (0,)),
        ],
        out_specs=pl.BlockSpec((blk, c), lambda i: (i, 0)),
    )(xf, yf, g, b, w1, b1, w2, b2, gamma)
    return out.reshape(n, h_, w_, c)


def kernel(currentBlock, eventCounts, hash_table, ds0_w, ds0_b, ln0_g, ln0_b,
           dw0_w, dw0_b, bln0_g, bln0_b, pw0_w1, pw0_b1, pw0_w2, pw0_b2,
           gamma0, ln1_g, ln1_b, ds1_w, ds1_b, dw1_w, dw1_b, bln1_g, bln1_b,
           pw1_w1, pw1_b1, pw1_w2, pw1_b2, gamma1, dec_w, dec_b, decln_g,
           decln_b, pred_w, pred_b):
    N = currentBlock.shape[0]
    coords = currentBlock[:, :3]
    res = _resolutions().astype(jnp.float32)
    primes = jnp.array([1, 2654435761, 805459861], dtype=jnp.uint32)
    feats = []
    for l in range(L):
        v = jnp.floor(coords * res[l]).astype(jnp.uint32)
        h = (v[:, 0] * primes[0]) ^ (v[:, 1] * primes[1]) ^ (v[:, 2] * primes[2])
        idx = (h % jnp.uint32(TS)).astype(jnp.int32)
        feats.append(jnp.take(hash_table[l], idx, axis=0))
    feats = jnp.concatenate(feats, axis=-1)
    cum = jnp.cumsum(eventCounts.astype(jnp.int32))
    bid = jnp.clip(jnp.searchsorted(cum, jnp.arange(N), side='right'), 0, B - 1)
    px = jnp.clip(jnp.round(currentBlock[:, 0] * W_).astype(jnp.int32), 0, W_ - 1)
    py = jnp.clip(jnp.round(currentBlock[:, 1] * H_).astype(jnp.int32), 0, H_ - 1)
    flat = bid * (W_ * H_) + px * H_ + py
    grid = jax.ops.segment_sum(feats, flat, num_segments=B * W_ * H_).reshape(B, W_, H_, L * F)

    x = _conv(grid, ds0_w, ds0_b, 2, 2)
    x = _ln(x, ln0_g, ln0_b)
    y = _conv(x, dw0_w, dw0_b, 1, 3, groups=96)
    x = _mlp_block(x, y, bln0_g, bln0_b, pw0_w1, pw0_b1, pw0_w2, pw0_b2, gamma0)
    x = _ln(x, ln1_g, ln1_b)
    x = _conv(x, ds1_w, ds1_b, 2, 2)
    y = _conv(x, dw1_w, dw1_b, 1, 2, groups=128)
    x = _mlp_block(x, y, bln1_g, bln1_b, pw1_w1, pw1_b1, pw1_w2, pw1_b2, gamma1)
    x = _conv(x, dec_w, dec_b, 1, 5, groups=128)
    x = _ln(x, decln_g, decln_b)
    x = jax.nn.relu(x)
    x = _conv(x, pred_w, pred_b, 1, 0)
    n, w, h, c = x.shape
    x = x.reshape(n, w, h, P, P, T_).transpose(0, 1, 3, 2, 4, 5).reshape(n, w * P, h * P, T_)
    return x
